# baseline (speedup 1.0000x reference)
import functools

import jax
import jax.numpy as jnp
from jax import lax
from jax.experimental import pallas as pl
from jax.experimental.pallas import tpu as pltpu
from jax.experimental.pallas import tpu_sc as plsc

DIM_ = 64
NEMB_ = 512
ROWS_ = 128 * 512
TC_BLOCK_ = 2048
NW_ = 32
ROWS_PER_W_ = ROWS_ // NW_
SC_CHUNK_ = 512
N_CHUNKS_ = ROWS_PER_W_ // SC_CHUNK_


def _tc_body(x_ref, e_ref, idx_ref, dsum_ref):
    i = pl.program_id(0)
    x = x_ref[...]
    e = e_ref[...]
    xe = jnp.dot(x, e, preferred_element_type=jnp.float32)
    dist = (
        jnp.sum(x * x, axis=1, keepdims=True)
        - 2.0 * xe
        + jnp.sum(e * e, axis=0, keepdims=True)
    )
    m = jnp.min(dist, axis=1, keepdims=True)
    jl = lax.broadcasted_iota(jnp.int32, (1, NEMB_), 1).astype(jnp.float32)
    masked = jnp.where(dist == m, jl, float(NEMB_))
    idx_ref[...] = jnp.min(masked, axis=1).astype(jnp.int32)

    @pl.when(i == 0)
    def _():
        dsum_ref[0, 0] = 0.0

    dsum_ref[0, 0] += jnp.sum(m)


def _tc_call(flat, embed):
    grid = ROWS_ // TC_BLOCK_
    return pl.pallas_call(
        _tc_body,
        grid=(grid,),
        in_specs=[
            pl.BlockSpec((TC_BLOCK_, DIM_), lambda i: (i, 0)),
            pl.BlockSpec((DIM_, NEMB_), lambda i: (0, 0)),
        ],
        out_specs=[
            pl.BlockSpec((TC_BLOCK_,), lambda i: (i,)),
            pl.BlockSpec(memory_space=pltpu.SMEM, block_shape=(1, 1),
                         index_map=lambda i: (0, 0)),
        ],
        out_shape=[
            jax.ShapeDtypeStruct((ROWS_,), jnp.int32),
            jax.ShapeDtypeStruct((1, 1), jnp.float32),
        ],
    )(flat, embed)


NBUF_ = 4
PAD_ = 128
SC_CHUNK_ = 128
N_CHUNKS_ = ROWS_PER_W_ // SC_CHUNK_


def _sc_gather_body(table_hbm, idx_hbm, out_hbm, idx_v, *bufs):
    rows = bufs[0:NBUF_]
    gsem = bufs[NBUF_:2 * NBUF_]
    osem = bufs[2 * NBUF_:3 * NBUF_]
    nc = 2
    wid = lax.axis_index("s") * nc + lax.axis_index("c")
    base = wid * ROWS_PER_W_
    pltpu.sync_copy(idx_hbm.at[wid], idx_v)
    gcp = [None] * N_CHUNKS_
    ocp = [None] * N_CHUNKS_
    for c in range(min(NBUF_, N_CHUNKS_)):
        gcp[c] = pltpu.async_copy(table_hbm.at[idx_v.at[c]], rows[c], gsem[c])
    for c in range(N_CHUNKS_):
        b = c % NBUF_
        gcp[c].wait()
        ocp[c] = pltpu.async_copy(
            rows[b], out_hbm.at[pl.ds(base + c * SC_CHUNK_, SC_CHUNK_)], osem[b])
        nxt = c + NBUF_
        if nxt < N_CHUNKS_:
            ocp[c].wait()
            gcp[nxt] = pltpu.async_copy(table_hbm.at[idx_v.at[nxt]], rows[b], gsem[b])
    for c in range(max(0, N_CHUNKS_ - NBUF_), N_CHUNKS_):
        ocp[c].wait()


@functools.cache
def _sc_gather():
    return pl.kernel(
        _sc_gather_body,
        out_type=jax.ShapeDtypeStruct((ROWS_, PAD_), jnp.float32),
        mesh=plsc.VectorSubcoreMesh(core_axis_name="c", subcore_axis_name="s"),
        scratch_types=(
            [pltpu.VMEM((N_CHUNKS_, SC_CHUNK_), jnp.int32)]
            + [pltpu.VMEM((SC_CHUNK_, PAD_), jnp.float32) for _ in range(NBUF_)]
            + [pltpu.SemaphoreType.DMA for _ in range(2 * NBUF_)]
        ),
    )


def kernel(inp, embed):
    flat = inp.reshape(ROWS_, DIM_)
    idx, dsum = _tc_call(flat, embed)
    table = jnp.concatenate(
        [embed.T, jnp.zeros((NEMB_, PAD_ - DIM_), jnp.float32)], axis=1)
    qp = _sc_gather()(table, idx.reshape(NW_, N_CHUNKS_, SC_CHUNK_))
    quantize_st = qp[:, :DIM_].reshape(128, 1, NEMB_, DIM_)
    diff = (1.25 / (ROWS_ * DIM_)) * dsum[0, 0]
    embed_ind = idx.reshape(128, 1, NEMB_)
    return quantize_st, diff, embed_ind


# speedup vs baseline: 2.2138x; 2.2138x over previous
import functools

import jax
import jax.numpy as jnp
from jax import lax
from jax.experimental import pallas as pl
from jax.experimental.pallas import tpu as pltpu
from jax.experimental.pallas import tpu_sc as plsc

DIM_ = 64
NEMB_ = 512
ROWS_ = 128 * 512
TC_BLOCK_ = 2048
NW_ = 32
ROWS_PER_W_ = ROWS_ // NW_
SC_CHUNK_ = 512
N_CHUNKS_ = ROWS_PER_W_ // SC_CHUNK_


def _tc_body(x_ref, e_ref, idx_ref, dsum_ref):
    i = pl.program_id(0)
    x = x_ref[...]
    e = e_ref[...]
    xe = jnp.dot(x, e, preferred_element_type=jnp.float32)
    dist = (
        jnp.sum(x * x, axis=1, keepdims=True)
        - 2.0 * xe
        + jnp.sum(e * e, axis=0, keepdims=True)
    )
    m = jnp.min(dist, axis=1, keepdims=True)
    jl = lax.broadcasted_iota(jnp.int32, (1, NEMB_), 1).astype(jnp.float32)
    masked = jnp.where(dist == m, jl, float(NEMB_))
    idx_ref[...] = jnp.min(masked, axis=1).astype(jnp.int32)

    @pl.when(i == 0)
    def _():
        dsum_ref[0, 0] = 0.0

    dsum_ref[0, 0] += jnp.sum(m)


def _tc_call(flat, embed):
    grid = ROWS_ // TC_BLOCK_
    return pl.pallas_call(
        _tc_body,
        grid=(grid,),
        in_specs=[
            pl.BlockSpec((TC_BLOCK_, DIM_), lambda i: (i, 0)),
            pl.BlockSpec((DIM_, NEMB_), lambda i: (0, 0)),
        ],
        out_specs=[
            pl.BlockSpec((TC_BLOCK_,), lambda i: (i,)),
            pl.BlockSpec(memory_space=pltpu.SMEM, block_shape=(1, 1),
                         index_map=lambda i: (0, 0)),
        ],
        out_shape=[
            jax.ShapeDtypeStruct((ROWS_,), jnp.int32),
            jax.ShapeDtypeStruct((1, 1), jnp.float32),
        ],
    )(flat, embed)


NBUF_ = 4
PAD_ = 128
SC_CHUNK_ = 128
N_CHUNKS_ = ROWS_PER_W_ // SC_CHUNK_


def _sc_gather_body(table_hbm, idx_hbm, out_hbm, table_sp, idx_v, *bufs):
    rows = bufs[0:NBUF_]
    gsem = bufs[NBUF_:2 * NBUF_]
    osem = bufs[2 * NBUF_:3 * NBUF_]
    nc = 2
    sid = lax.axis_index("s")
    wid = sid * nc + lax.axis_index("c")
    base = wid * ROWS_PER_W_

    @pl.when(sid == 0)
    def _():
        pltpu.sync_copy(table_hbm, table_sp)

    pltpu.sync_copy(idx_hbm.at[wid], idx_v)
    plsc.subcore_barrier()
    gcp = [None] * N_CHUNKS_
    ocp = [None] * N_CHUNKS_
    for c in range(min(NBUF_, N_CHUNKS_)):
        gcp[c] = pltpu.async_copy(table_sp.at[idx_v.at[c]], rows[c], gsem[c])
    for c in range(N_CHUNKS_):
        b = c % NBUF_
        gcp[c].wait()
        ocp[c] = pltpu.async_copy(
            rows[b], out_hbm.at[pl.ds(base + c * SC_CHUNK_, SC_CHUNK_)], osem[b])
        nxt = c + NBUF_
        if nxt < N_CHUNKS_:
            ocp[c].wait()
            gcp[nxt] = pltpu.async_copy(table_sp.at[idx_v.at[nxt]], rows[b], gsem[b])
    for c in range(max(0, N_CHUNKS_ - NBUF_), N_CHUNKS_):
        ocp[c].wait()


@functools.cache
def _sc_gather():
    return pl.kernel(
        _sc_gather_body,
        out_type=jax.ShapeDtypeStruct((ROWS_, PAD_), jnp.float32),
        mesh=plsc.VectorSubcoreMesh(core_axis_name="c", subcore_axis_name="s"),
        scratch_types=(
            [pltpu.VMEM_SHARED((NEMB_, PAD_), jnp.float32)]
            + [pltpu.VMEM((N_CHUNKS_, SC_CHUNK_), jnp.int32)]
            + [pltpu.VMEM((SC_CHUNK_, PAD_), jnp.float32) for _ in range(NBUF_)]
            + [pltpu.SemaphoreType.DMA for _ in range(2 * NBUF_)]
        ),
    )


def kernel(inp, embed):
    flat = inp.reshape(ROWS_, DIM_)
    idx, dsum = _tc_call(flat, embed)
    table = jnp.concatenate(
        [embed.T, jnp.zeros((NEMB_, PAD_ - DIM_), jnp.float32)], axis=1)
    qp = _sc_gather()(table, idx.reshape(NW_, N_CHUNKS_, SC_CHUNK_))
    quantize_st = qp[:, :DIM_].reshape(128, 1, NEMB_, DIM_)
    diff = (1.25 / (ROWS_ * DIM_)) * dsum[0, 0]
    embed_ind = idx.reshape(128, 1, NEMB_)
    return quantize_st, diff, embed_ind
